# baseline (speedup 1.0000x reference)
import functools

import jax
import jax.numpy as jnp
from jax.experimental import pallas as pl
from jax.experimental.pallas import tpu as pltpu

_TM = 2048
_TV = 512


def _fused_mlp_kernel(x_ref, w1_ref, b1_ref, w2_ref, b2_ref, s_ref, o_ref,
                      xb_ref, *, n_v):
    v = pl.program_id(1)

    @pl.when(v == 0)
    def _():
        xb_ref[...] = x_ref[...].astype(jnp.bfloat16)

    h = jnp.dot(xb_ref[...], w1_ref[...].astype(jnp.bfloat16),
                preferred_element_type=jnp.float32)
    h = jnp.maximum(h + b1_ref[...], 0.0).astype(jnp.bfloat16)
    p = jnp.dot(h, w2_ref[...].astype(jnp.bfloat16),
                preferred_element_type=jnp.float32)

    @pl.when(v == 0)
    def _():
        o_ref[...] = p

    @pl.when(jnp.logical_and(v > 0, v < n_v - 1))
    def _():
        o_ref[...] += p

    @pl.when(v == n_v - 1)
    def _():
        y = o_ref[...] + p + b2_ref[...] + x_ref[...]
        var = jnp.mean(y * y, axis=-1, keepdims=True)
        o_ref[...] = (y * jax.lax.rsqrt(var + 1e-6)) * s_ref[...]


def kernel(x, W1, b1, W2, b2, ln_scale):
    B, S, D = x.shape
    K, V = W1.shape
    M = B * S
    n_m = M // _TM
    n_v = V // _TV

    xf = x.reshape(M, D)
    b1r = b1.reshape(1, V)
    b2r = b2.reshape(1, D)
    snr = ln_scale.reshape(1, D)

    body = functools.partial(_fused_mlp_kernel, n_v=n_v)

    out = pl.pallas_call(
        body,
        grid=(n_m, n_v),
        in_specs=[
            pl.BlockSpec((_TM, D), lambda m, v: (m, 0)),
            pl.BlockSpec((K, _TV), lambda m, v: (0, v)),
            pl.BlockSpec((1, _TV), lambda m, v: (0, v)),
            pl.BlockSpec((_TV, D), lambda m, v: (v, 0)),
            pl.BlockSpec((1, D), lambda m, v: (0, 0)),
            pl.BlockSpec((1, D), lambda m, v: (0, 0)),
        ],
        out_specs=pl.BlockSpec((_TM, D), lambda m, v: (m, 0)),
        out_shape=jax.ShapeDtypeStruct((M, D), jnp.float32),
        scratch_shapes=[pltpu.VMEM((_TM, D), jnp.bfloat16)],
        compiler_params=pltpu.CompilerParams(
            dimension_semantics=("parallel", "arbitrary"),
        ),
    )(xf, W1, b1r, W2, b2r, snr)
    return out.reshape(B, S, D)


# speedup vs baseline: 1.1533x; 1.1533x over previous
import jax
import jax.numpy as jnp
from jax.experimental import pallas as pl
from jax.experimental.pallas import tpu as pltpu

_TM = 512


def _fused_mlp_kernel(x_ref, w1_ref, b1_ref, w2_ref, b2_ref, s_ref, o_ref):
    x = x_ref[...]
    h = jnp.dot(x.astype(jnp.bfloat16), w1_ref[...].astype(jnp.bfloat16),
                preferred_element_type=jnp.float32)
    h = jnp.maximum(h + b1_ref[...], 0.0).astype(jnp.bfloat16)
    y = jnp.dot(h, w2_ref[...].astype(jnp.bfloat16),
                preferred_element_type=jnp.float32)
    y = y + b2_ref[...] + x
    var = jnp.mean(y * y, axis=-1, keepdims=True)
    o_ref[...] = (y * jax.lax.rsqrt(var + 1e-6)) * s_ref[...]


def kernel(x, W1, b1, W2, b2, ln_scale):
    B, S, D = x.shape
    K, V = W1.shape
    M = B * S

    xf = x.reshape(M, D)
    b1r = b1.reshape(1, V)
    b2r = b2.reshape(1, D)
    snr = ln_scale.reshape(1, D)

    out = pl.pallas_call(
        _fused_mlp_kernel,
        grid=(M // _TM,),
        in_specs=[
            pl.BlockSpec((_TM, D), lambda m: (m, 0)),
            pl.BlockSpec((K, V), lambda m: (0, 0)),
            pl.BlockSpec((1, V), lambda m: (0, 0)),
            pl.BlockSpec((V, D), lambda m: (0, 0)),
            pl.BlockSpec((1, D), lambda m: (0, 0)),
            pl.BlockSpec((1, D), lambda m: (0, 0)),
        ],
        out_specs=pl.BlockSpec((_TM, D), lambda m: (m, 0)),
        out_shape=jax.ShapeDtypeStruct((M, D), jnp.float32),
        compiler_params=pltpu.CompilerParams(
            dimension_semantics=("parallel",),
        ),
    )(xf, W1, b1r, W2, b2r, snr)
    return out.reshape(B, S, D)
